# Initial kernel scaffold; baseline (speedup 1.0000x reference)
import jax
import jax.numpy as jnp
from jax.experimental import pallas as pl


def kernel(U, rho, nodIdx, filters, typeFilter):
    raise NotImplementedError("write your pallas kernel here")


# speedup vs baseline: 16.7071x; 16.7071x over previous
import jax
import jax.numpy as jnp
from jax import lax
from jax.experimental import pallas as pl
from jax.experimental.pallas import tpu as pltpu
from jax.experimental.pallas import tpu_sc as plsc

N_NODES_C = 97336
N3 = N_NODES_C * 3
SL = 18256
N3_PAD = SL * 16
E_C = 91125
NW = 32
CHUNK = 2880
E_PAD = NW * CHUNK
SUB = 480
NSUB = CHUNK // SUB
NBLK = SUB // 16
NT = 16
KD = 24


def _sc_body(u_hbm, nod_hbm, rho_hbm, filt_hbm, tf_hbm, part_hbm,
             filt_v, nod_v, rho_v, idx_v, ue_v, fe_v, tf_v, zb_v, ku_sh,
             gsem):
    cid = lax.axis_index("c")
    sid = lax.axis_index("s")
    wid = sid * 2 + cid

    pltpu.sync_copy(filt_hbm, filt_v)
    pltpu.sync_copy(tf_hbm, tf_v)
    scale = tf_v[...]

    zeros16 = jnp.zeros((16,), jnp.float32)

    @pl.loop(0, SL // 16)
    def _zero(i):
        zb_v[pl.ds(i * 16, 16)] = zeros16

    pltpu.sync_copy(zb_v, ku_sh.at[pl.ds(sid * SL, SL)])
    plsc.subcore_barrier()

    chunk_base = wid * CHUNK

    @pl.loop(0, NSUB)
    def _sub(s):
        base = chunk_base + s * SUB
        for j in range(8):
            pltpu.sync_copy(nod_hbm.at[pl.ds(j * E_PAD + base, SUB)],
                            nod_v.at[pl.ds(j * SUB, SUB)])
        pltpu.sync_copy(rho_hbm.at[pl.ds(base, SUB)], rho_v)

        @pl.loop(0, NBLK)
        def _bidx(b):
            for j in range(8):
                nj = nod_v[pl.ds(j * SUB + b * 16, 16)]
                n3 = nj * 3
                for k in range(3):
                    idx_v[pl.ds(b * (KD * 16) + (j * 3 + k) * 16, 16)] = n3 + k

        pltpu.async_copy(u_hbm.at[idx_v], ue_v, gsem).wait()

        @pl.loop(0, NBLK)
        def _blk(b):
            rv = rho_v[pl.ds(b * 16, 16)]
            tv = (rv * scale).astype(jnp.int32)
            tv = jnp.clip(lax.rem(tv, NT), 0, NT - 1)
            toff = tv * (KD * KD)
            us = [ue_v[pl.ds(b * (KD * 16) + jj * 16, 16)] for jj in range(KD)]
            for i in range(KD):
                a = [None, None, None, None]
                for j in range(KD):
                    wv = plsc.load_gather(filt_v, [toff + (i * KD + j)])
                    t = wv * us[j]
                    a[j % 4] = t if a[j % 4] is None else a[j % 4] + t
                fi = ((a[0] + a[1]) + (a[2] + a[3])) * rv
                fe_v[pl.ds(b * (KD * 16) + i * 16, 16)] = fi

        pltpu.sync_copy(fe_v, ku_sh.at[idx_v], add=True)

    plsc.subcore_barrier()
    pltpu.sync_copy(ku_sh.at[pl.ds(sid * SL, SL)], zb_v)
    pltpu.sync_copy(zb_v, part_hbm.at[pl.ds(cid * N3_PAD + sid * SL, SL)])


_sc_call = pl.kernel(
    _sc_body,
    out_type=jax.ShapeDtypeStruct((2 * N3_PAD,), jnp.float32),
    mesh=plsc.VectorSubcoreMesh(core_axis_name="c", subcore_axis_name="s"),
    scratch_types=[
        pltpu.VMEM((NT * KD * KD,), jnp.float32),
        pltpu.VMEM((8 * SUB,), jnp.int32),
        pltpu.VMEM((SUB,), jnp.float32),
        pltpu.VMEM((SUB * KD,), jnp.int32),
        pltpu.VMEM((SUB * KD,), jnp.float32),
        pltpu.VMEM((SUB * KD,), jnp.float32),
        pltpu.VMEM((16,), jnp.float32),
        pltpu.VMEM((SL,), jnp.float32),
        pltpu.VMEM_SHARED((N3_PAD,), jnp.float32),
        pltpu.SemaphoreType.DMA,
    ],
    compiler_params=pltpu.CompilerParams(needs_layout_passes=False),
)


def _sum_body(p_ref, o_ref):
    o_ref[...] = p_ref[0] + p_ref[1]


def _tc_sum(part):
    p3 = part.reshape(2, N3_PAD // 128, 128)
    return pl.pallas_call(
        _sum_body,
        out_shape=jax.ShapeDtypeStruct((N3_PAD // 128, 128), jnp.float32),
    )(p3)


def kernel(U, rho, nodIdx, filters, typeFilter):
    Uf = U.reshape(-1)
    nodT = jnp.concatenate(
        [nodIdx.T, jnp.zeros((8, E_PAD - E_C), jnp.int32)], axis=1).reshape(-1)
    rho_p = jnp.concatenate([rho, jnp.zeros((E_PAD - E_C,), jnp.float32)])
    filt_f = filters.reshape(-1)
    tf16 = jnp.full((16,), jnp.sum(typeFilter), dtype=jnp.float32)
    part = _sc_call(Uf, nodT, rho_p, filt_f, tf16)
    s = _tc_sum(part)
    KU = s.reshape(-1)[:N3].reshape(N_NODES_C, 3)
    return KU, U


# speedup vs baseline: 33.3080x; 1.9936x over previous
import jax
import jax.numpy as jnp
from jax import lax
from jax.experimental import pallas as pl
from jax.experimental.pallas import tpu as pltpu
from jax.experimental.pallas import tpu_sc as plsc

N_NODES_C = 97336
N3 = N_NODES_C * 3
SL = 18256
N3_PAD = SL * 16
E_C = 91125
NW = 32
CHUNK = 2880
E_PAD = NW * CHUNK
SUB = 480
NSUB = CHUNK // SUB
NBLK = SUB // 16
NT = 16
KD = 24


def _sc_body(u_hbm, nod_hbm, rho_hbm, filt_hbm, tf_hbm, part_hbm,
             filt_v, nod_v, rho_v, idx_v, ue_v, fe_v, tf_v, zb_v, ku_sh,
             gsem):
    cid = lax.axis_index("c")
    sid = lax.axis_index("s")
    wid = sid * 2 + cid

    pltpu.sync_copy(filt_hbm, filt_v)
    pltpu.sync_copy(tf_hbm, tf_v)
    scale = tf_v[...]

    zeros16 = jnp.zeros((16,), jnp.float32)

    @pl.loop(0, SL // 16)
    def _zero(i):
        zb_v[pl.ds(i * 16, 16)] = zeros16

    pltpu.sync_copy(zb_v, ku_sh.at[pl.ds(sid * SL, SL)])
    plsc.subcore_barrier()

    chunk_base = wid * CHUNK

    @pl.loop(0, NSUB)
    def _sub(s):
        base = chunk_base + s * SUB
        for j in range(8):
            pltpu.sync_copy(nod_hbm.at[pl.ds(j * E_PAD + base, SUB)],
                            nod_v.at[pl.ds(j * SUB, SUB)])
        pltpu.sync_copy(rho_hbm.at[pl.ds(base, SUB)], rho_v)

        @pl.loop(0, NBLK)
        def _bidx(b):
            for j in range(8):
                nj = nod_v[pl.ds(j * SUB + b * 16, 16)]
                n3 = nj * 3
                for k in range(3):
                    idx_v[pl.ds(b * (KD * 16) + (j * 3 + k) * 16, 16)] = n3 + k

        pltpu.async_copy(u_hbm.at[idx_v], ue_v, gsem).wait()

        @pl.loop(0, NBLK)
        def _blk(b):
            rv = rho_v[pl.ds(b * 16, 16)]
            tv = (rv * scale).astype(jnp.int32)
            tv = jnp.clip(lax.rem(tv, NT), 0, NT - 1)
            us = [ue_v[pl.ds(b * (KD * 16) + jj * 16, 16)] for jj in range(KD)]
            for i in range(KD):
                a = [None, None, None, None]
                for j in range(KD):
                    wv = plsc.load_gather(
                        filt_v.at[pl.ds((i * KD + j) * NT, NT)], [tv])
                    t = wv * us[j]
                    a[j % 4] = t if a[j % 4] is None else a[j % 4] + t
                fi = ((a[0] + a[1]) + (a[2] + a[3])) * rv
                fe_v[pl.ds(b * (KD * 16) + i * 16, 16)] = fi

        pltpu.sync_copy(fe_v, ku_sh.at[idx_v], add=True)

    plsc.subcore_barrier()
    pltpu.sync_copy(ku_sh.at[pl.ds(sid * SL, SL)], zb_v)
    pltpu.sync_copy(zb_v, part_hbm.at[pl.ds(cid * N3_PAD + sid * SL, SL)])


_sc_call = pl.kernel(
    _sc_body,
    out_type=jax.ShapeDtypeStruct((2 * N3_PAD,), jnp.float32),
    mesh=plsc.VectorSubcoreMesh(core_axis_name="c", subcore_axis_name="s"),
    scratch_types=[
        pltpu.VMEM((NT * KD * KD,), jnp.float32),
        pltpu.VMEM((8 * SUB,), jnp.int32),
        pltpu.VMEM((SUB,), jnp.float32),
        pltpu.VMEM((SUB * KD,), jnp.int32),
        pltpu.VMEM((SUB * KD,), jnp.float32),
        pltpu.VMEM((SUB * KD,), jnp.float32),
        pltpu.VMEM((16,), jnp.float32),
        pltpu.VMEM((SL,), jnp.float32),
        pltpu.VMEM_SHARED((N3_PAD,), jnp.float32),
        pltpu.SemaphoreType.DMA,
    ],
    compiler_params=pltpu.CompilerParams(needs_layout_passes=False),
)


def _sum_body(p_ref, o_ref):
    o_ref[...] = p_ref[0] + p_ref[1]


def _tc_sum(part):
    p3 = part.reshape(2, N3_PAD // 128, 128)
    return pl.pallas_call(
        _sum_body,
        out_shape=jax.ShapeDtypeStruct((N3_PAD // 128, 128), jnp.float32),
    )(p3)


def kernel(U, rho, nodIdx, filters, typeFilter):
    Uf = U.reshape(-1)
    nodT = jnp.concatenate(
        [nodIdx.T, jnp.zeros((8, E_PAD - E_C), jnp.int32)], axis=1).reshape(-1)
    rho_p = jnp.concatenate([rho, jnp.zeros((E_PAD - E_C,), jnp.float32)])
    filt_f = filters.reshape(NT, KD * KD).T.reshape(-1)
    tf16 = jnp.full((16,), jnp.sum(typeFilter), dtype=jnp.float32)
    part = _sc_call(Uf, nodT, rho_p, filt_f, tf16)
    s = _tc_sum(part)
    KU = s.reshape(-1)[:N3].reshape(N_NODES_C, 3)
    return KU, U
